# baseline (speedup 1.0000x reference)
import functools

import jax
import jax.numpy as jnp
from jax import lax
from jax.experimental import pallas as pl
from jax.experimental.pallas import tpu as pltpu
from jax.experimental.pallas import tpu_sc as plsc

BATCH = 1024
SEQ_LEN = 200
EMB_DIM = 32
VOCAB = 1000000
OUT_ROWS = SEQ_LEN + 1
CHUNKS = (104, 96)
GROUP = 8

_info = plsc.get_sparse_core_info()
_NC, _NS = _info.num_cores, _info.num_subcores
NW = _NC * _NS
BPW = BATCH // NW
NGROUP = BPW // GROUP
BLOCK_ROWS = GROUP * OUT_ROWS

TCHUNK = 256
NCHUNK = VOCAB // TCHUNK
TAIL = VOCAB - NCHUNK * TCHUNK
VPAD = VOCAB + (-VOCAB % TCHUNK)
TWORDS = VPAD * EMB_DIM


@functools.partial(
    pl.kernel,
    mesh=plsc.VectorSubcoreMesh(core_axis_name="c", subcore_axis_name="s"),
    out_type=jax.ShapeDtypeStruct((TWORDS,), jnp.float32),
    scratch_types=[
        pltpu.VMEM((EMB_DIM, TCHUNK), jnp.float32),
        pltpu.VMEM((EMB_DIM, TCHUNK), jnp.float32),
        pltpu.VMEM((TCHUNK * EMB_DIM,), jnp.float32),
        pltpu.VMEM((TCHUNK * EMB_DIM,), jnp.float32),
        pltpu.VMEM((TAIL * EMB_DIM,), jnp.float32),
        pltpu.SemaphoreType.DMA,
        pltpu.SemaphoreType.DMA,
        pltpu.SemaphoreType.DMA,
        pltpu.SemaphoreType.DMA,
    ],
    compiler_params=pltpu.CompilerParams(use_tc_tiling_on_sc=True,
                                        needs_layout_passes=False),
)
def _detile_kernel(tt_hbm, tail_hbm, out_hbm,
                   in_a, in_b, tr_a, tr_b, tail_v,
                   sem_ia, sem_ib, sem_oa, sem_ob):
    wid = lax.axis_index("s") * _NC + lax.axis_index("c")
    lane = lax.iota(jnp.int32, 16)

    @pl.when(wid == 0)
    def _():
        pltpu.sync_copy(tail_hbm, tail_v)
        pltpu.sync_copy(tail_v,
                        out_hbm.at[pl.ds(NCHUNK * TCHUNK * EMB_DIM,
                                         TAIL * EMB_DIM)])

    def in_desc(t, in_x, sem):
        r0 = (wid + NW * t) * TCHUNK
        return pltpu.make_async_copy(
            tt_hbm.at[pl.ds(0, EMB_DIM), pl.ds(r0, TCHUNK)], in_x, sem)

    def out_desc(t, tr_x, sem):
        cc = wid + NW * t
        return pltpu.make_async_copy(
            tr_x, out_hbm.at[pl.ds(cc * TCHUNK * EMB_DIM, TCHUNK * EMB_DIM)],
            sem)

    lane32 = lane * EMB_DIM

    def transpose(in_x, tr_x):
        @plsc.parallel_loop(0, EMB_DIM, step=1, unroll=2)
        def tbody(c):
            for m in range(TCHUNK // 16):
                vals = in_x[c, pl.ds(16 * m, 16)]
                plsc.store_scatter(tr_x, [lane32 + (512 * m + c)], vals)

    bufs = ((in_a, tr_a, sem_ia, sem_oa), (in_b, tr_b, sem_ib, sem_ob))
    in_desc(0, in_a, sem_ia).start()
    in_desc(1, in_b, sem_ib).start()

    def body(tau, _):
        for par, (in_x, tr_x, sem_i, sem_o) in enumerate(bufs):
            t = 2 * tau + par

            @pl.when(wid + NW * t < NCHUNK)
            def _():
                in_desc(t, in_x, sem_i).wait()

                @pl.when(t >= 2)
                def _():
                    out_desc(t - 2, tr_x, sem_o).wait()

                transpose(in_x, tr_x)
                out_desc(t, tr_x, sem_o).start()

                @pl.when(wid + NW * (t + 2) < NCHUNK)
                def _():
                    in_desc(t + 2, in_x, sem_i).start()

        return 0

    tmax = (NCHUNK + NW - 1) // NW
    lax.fori_loop(0, (tmax + 2) // 2, body, 0)

    n_w = (NCHUNK - wid + NW - 1) // NW
    for par, (in_x, tr_x, sem_i, sem_o) in enumerate(bufs):
        t_last = par + 2 * ((n_w - par - 1) // 2)
        out_desc(t_last, tr_x, sem_o).wait()


@functools.partial(
    pl.kernel,
    mesh=plsc.VectorSubcoreMesh(core_axis_name="c", subcore_axis_name="s"),
    out_type=jax.ShapeDtypeStruct((BATCH, OUT_ROWS, EMB_DIM), jnp.float32),
    scratch_types=[
        pltpu.VMEM((BPW * SEQ_LEN,), jnp.int32),
        pltpu.VMEM((2 * GROUP, OUT_ROWS, EMB_DIM), jnp.float32),
        pltpu.SemaphoreType.DMA,
        pltpu.SemaphoreType.DMA,
    ],
    compiler_params=pltpu.CompilerParams(use_tc_tiling_on_sc=False),
)
def _emb_kernel(idx_hbm, table_hbm, empty_hbm, out_hbm,
                idx_v, rows_v, sem_g, sem_s):
    wid = lax.axis_index("s") * _NC + lax.axis_index("c")
    base = wid * BPW
    pltpu.sync_copy(idx_hbm.at[pl.ds(base * SEQ_LEN, BPW * SEQ_LEN)], idx_v)
    for d in range(2):
        for j in range(GROUP):
            pltpu.sync_copy(
                empty_hbm, rows_v.at[d * GROUP + j, pl.ds(0, 1)])

    def gather_copies(g, d, make_only):
        for j in range(GROUP):
            off = 0
            for n in CHUNKS:
                desc = pltpu.make_async_copy(
                    table_hbm.at[idx_v.at[pl.ds((g * GROUP + j) * SEQ_LEN
                                                + off, n)]],
                    rows_v.at[d * GROUP + j, pl.ds(1 + off, n)],
                    sem_g)
                if make_only:
                    desc.wait()
                else:
                    desc.start()
                off += n

    def gather_group(g, d):
        gather_copies(g, d, make_only=False)

    def drain_gathers(g, d):
        gather_copies(g, d, make_only=True)

    def store_desc(g, d):
        return pltpu.make_async_copy(
            rows_v.at[pl.ds(d * GROUP, GROUP)],
            out_hbm.at[pl.ds(base + g * GROUP, GROUP)],
            sem_s)

    gather_group(0, 0)

    def body(g, _):
        d = g % 2
        drain_gathers(g, d)

        @pl.when(g >= 1)
        def _():
            store_desc(g - 1, 1 - d).wait()

        store_desc(g, d).start()

        @pl.when(g + 1 < NGROUP)
        def _():
            gather_group(g + 1, 1 - d)

        return 0

    lax.fori_loop(0, NGROUP, body, 0)
    store_desc(NGROUP - 1, (NGROUP - 1) % 2).wait()


def kernel(sentence, table, empty_emb):
    idx = sentence[:, ::-1].astype(jnp.int32).reshape(-1)
    tail = table[NCHUNK * TCHUNK:].reshape(-1)
    table_lin = _detile_kernel(table.T, tail)
    table_rm = table_lin.reshape(VPAD, EMB_DIM)
    return _emb_kernel(idx, table_rm, empty_emb)


# speedup vs baseline: 1.2420x; 1.2420x over previous
import functools

import jax
import jax.numpy as jnp
from jax import lax
from jax.experimental import pallas as pl
from jax.experimental.pallas import tpu as pltpu
from jax.experimental.pallas import tpu_sc as plsc

BATCH = 1024
SEQ_LEN = 200
EMB_DIM = 32
VOCAB = 1000000
OUT_ROWS = SEQ_LEN + 1
CHUNKS = (104, 96)
GROUP = 8

_info = plsc.get_sparse_core_info()
_NC, _NS = _info.num_cores, _info.num_subcores
NW = _NC * _NS
BPW = BATCH // NW
NGROUP = BPW // GROUP
BLOCK_ROWS = GROUP * OUT_ROWS

TCHUNK = 256
NCHUNK = VOCAB // TCHUNK
TAIL = VOCAB - NCHUNK * TCHUNK
VPAD = VOCAB + (-VOCAB % TCHUNK)
TWORDS = VPAD * EMB_DIM


@functools.partial(
    pl.kernel,
    mesh=plsc.VectorSubcoreMesh(core_axis_name="c", subcore_axis_name="s"),
    out_type=jax.ShapeDtypeStruct((TWORDS,), jnp.float32),
    scratch_types=[
        pltpu.VMEM((EMB_DIM, TCHUNK), jnp.float32),
        pltpu.VMEM((EMB_DIM, TCHUNK), jnp.float32),
        pltpu.VMEM((TCHUNK * EMB_DIM,), jnp.float32),
        pltpu.VMEM((TCHUNK * EMB_DIM,), jnp.float32),
        pltpu.VMEM((TAIL * EMB_DIM,), jnp.float32),
        pltpu.SemaphoreType.DMA,
        pltpu.SemaphoreType.DMA,
        pltpu.SemaphoreType.DMA,
        pltpu.SemaphoreType.DMA,
    ],
    compiler_params=pltpu.CompilerParams(use_tc_tiling_on_sc=True,
                                        needs_layout_passes=False),
)
def _detile_kernel(tt_hbm, tail_hbm, out_hbm,
                   in_a, in_b, tr_a, tr_b, tail_v,
                   sem_ia, sem_ib, sem_oa, sem_ob):
    wid = lax.axis_index("s") * _NC + lax.axis_index("c")
    lane = lax.iota(jnp.int32, 16)

    @pl.when(wid == 0)
    def _():
        pltpu.sync_copy(tail_hbm, tail_v)
        pltpu.sync_copy(tail_v,
                        out_hbm.at[pl.ds(NCHUNK * TCHUNK * EMB_DIM,
                                         TAIL * EMB_DIM)])

    def in_desc(t, in_x, sem):
        r0 = (wid + NW * t) * TCHUNK
        return pltpu.make_async_copy(
            tt_hbm.at[pl.ds(0, EMB_DIM), pl.ds(r0, TCHUNK)], in_x, sem)

    def out_desc(t, tr_x, sem):
        cc = wid + NW * t
        return pltpu.make_async_copy(
            tr_x, out_hbm.at[pl.ds(cc * TCHUNK * EMB_DIM, TCHUNK * EMB_DIM)],
            sem)

    rots = [(lane + s) % 16 for s in range(16)]
    stxs = [r * EMB_DIM + lane for r in rots]

    def transpose(in_x, tr_x):
        @plsc.parallel_loop(0, TCHUNK // 16, step=1, unroll=2)
        def tbody(m):
            for h in range(EMB_DIM // 16):
                for s in range(16):
                    vals = plsc.load_gather(
                        in_x, [lane + 16 * h, rots[s] + 16 * m])
                    plsc.store_scatter(
                        tr_x, [stxs[s] + (512 * m + 16 * h)], vals)

    bufs = ((in_a, tr_a, sem_ia, sem_oa), (in_b, tr_b, sem_ib, sem_ob))
    in_desc(0, in_a, sem_ia).start()
    in_desc(1, in_b, sem_ib).start()

    def body(tau, _):
        for par, (in_x, tr_x, sem_i, sem_o) in enumerate(bufs):
            t = 2 * tau + par

            @pl.when(wid + NW * t < NCHUNK)
            def _():
                in_desc(t, in_x, sem_i).wait()

                @pl.when(t >= 2)
                def _():
                    out_desc(t - 2, tr_x, sem_o).wait()

                transpose(in_x, tr_x)
                out_desc(t, tr_x, sem_o).start()

                @pl.when(wid + NW * (t + 2) < NCHUNK)
                def _():
                    in_desc(t + 2, in_x, sem_i).start()

        return 0

    tmax = (NCHUNK + NW - 1) // NW
    lax.fori_loop(0, (tmax + 2) // 2, body, 0)

    n_w = (NCHUNK - wid + NW - 1) // NW
    for par, (in_x, tr_x, sem_i, sem_o) in enumerate(bufs):
        t_last = par + 2 * ((n_w - par - 1) // 2)
        out_desc(t_last, tr_x, sem_o).wait()


@functools.partial(
    pl.kernel,
    mesh=plsc.VectorSubcoreMesh(core_axis_name="c", subcore_axis_name="s"),
    out_type=jax.ShapeDtypeStruct((BATCH, OUT_ROWS, EMB_DIM), jnp.float32),
    scratch_types=[
        pltpu.VMEM((BPW * SEQ_LEN,), jnp.int32),
        pltpu.VMEM((2 * GROUP, OUT_ROWS, EMB_DIM), jnp.float32),
        pltpu.SemaphoreType.DMA,
        pltpu.SemaphoreType.DMA,
    ],
    compiler_params=pltpu.CompilerParams(use_tc_tiling_on_sc=False),
)
def _emb_kernel(idx_hbm, table_hbm, empty_hbm, out_hbm,
                idx_v, rows_v, sem_g, sem_s):
    wid = lax.axis_index("s") * _NC + lax.axis_index("c")
    base = wid * BPW
    pltpu.sync_copy(idx_hbm.at[pl.ds(base * SEQ_LEN, BPW * SEQ_LEN)], idx_v)
    for d in range(2):
        for j in range(GROUP):
            pltpu.sync_copy(
                empty_hbm, rows_v.at[d * GROUP + j, pl.ds(0, 1)])

    def gather_copies(g, d, make_only):
        for j in range(GROUP):
            off = 0
            for n in CHUNKS:
                desc = pltpu.make_async_copy(
                    table_hbm.at[idx_v.at[pl.ds((g * GROUP + j) * SEQ_LEN
                                                + off, n)]],
                    rows_v.at[d * GROUP + j, pl.ds(1 + off, n)],
                    sem_g)
                if make_only:
                    desc.wait()
                else:
                    desc.start()
                off += n

    def gather_group(g, d):
        gather_copies(g, d, make_only=False)

    def drain_gathers(g, d):
        gather_copies(g, d, make_only=True)

    def store_desc(g, d):
        return pltpu.make_async_copy(
            rows_v.at[pl.ds(d * GROUP, GROUP)],
            out_hbm.at[pl.ds(base + g * GROUP, GROUP)],
            sem_s)

    gather_group(0, 0)

    def body(g, _):
        d = g % 2
        drain_gathers(g, d)

        @pl.when(g >= 1)
        def _():
            store_desc(g - 1, 1 - d).wait()

        store_desc(g, d).start()

        @pl.when(g + 1 < NGROUP)
        def _():
            gather_group(g + 1, 1 - d)

        return 0

    lax.fori_loop(0, NGROUP, body, 0)
    store_desc(NGROUP - 1, (NGROUP - 1) % 2).wait()


def kernel(sentence, table, empty_emb):
    idx = sentence[:, ::-1].astype(jnp.int32).reshape(-1)
    tail = table[NCHUNK * TCHUNK:].reshape(-1)
    table_lin = _detile_kernel(table.T, tail)
    table_rm = table_lin.reshape(VPAD, EMB_DIM)
    return _emb_kernel(idx, table_rm, empty_emb)


# speedup vs baseline: 1.2810x; 1.0314x over previous
import functools

import jax
import jax.numpy as jnp
from jax import lax
from jax.experimental import pallas as pl
from jax.experimental.pallas import tpu as pltpu
from jax.experimental.pallas import tpu_sc as plsc

BATCH = 1024
SEQ_LEN = 200
EMB_DIM = 32
VOCAB = 1000000
OUT_ROWS = SEQ_LEN + 1
CHUNKS = (104, 96)
GROUP = 8

_info = plsc.get_sparse_core_info()
_NC, _NS = _info.num_cores, _info.num_subcores
NW = _NC * _NS
BPW = BATCH // NW
NGROUP = BPW // GROUP
BLOCK_ROWS = GROUP * OUT_ROWS

TCHUNK = 256
NCHUNK = VOCAB // TCHUNK
TAIL = VOCAB - NCHUNK * TCHUNK
VPAD = VOCAB + (-VOCAB % TCHUNK)
TWORDS = VPAD * EMB_DIM


@functools.partial(
    pl.kernel,
    mesh=plsc.VectorSubcoreMesh(core_axis_name="c", subcore_axis_name="s"),
    out_type=jax.ShapeDtypeStruct((TWORDS,), jnp.float32),
    scratch_types=[
        pltpu.VMEM((EMB_DIM, TCHUNK), jnp.float32),
        pltpu.VMEM((EMB_DIM, TCHUNK), jnp.float32),
        pltpu.VMEM((TCHUNK * EMB_DIM,), jnp.float32),
        pltpu.VMEM((TCHUNK * EMB_DIM,), jnp.float32),
        pltpu.VMEM((TAIL * EMB_DIM,), jnp.float32),
        pltpu.SemaphoreType.DMA,
        pltpu.SemaphoreType.DMA,
        pltpu.SemaphoreType.DMA,
        pltpu.SemaphoreType.DMA,
    ],
    compiler_params=pltpu.CompilerParams(use_tc_tiling_on_sc=True,
                                        needs_layout_passes=False),
)
def _detile_kernel(tt_hbm, tail_hbm, out_hbm,
                   in_a, in_b, tr_a, tr_b, tail_v,
                   sem_ia, sem_ib, sem_oa, sem_ob):
    wid = lax.axis_index("s") * _NC + lax.axis_index("c")
    lane = lax.iota(jnp.int32, 16)

    @pl.when(wid == 0)
    def _():
        pltpu.sync_copy(tail_hbm, tail_v)
        pltpu.sync_copy(tail_v,
                        out_hbm.at[pl.ds(NCHUNK * TCHUNK * EMB_DIM,
                                         TAIL * EMB_DIM)])

    def in_desc(t, in_x, sem):
        r0 = (wid + NW * t) * TCHUNK
        return pltpu.make_async_copy(
            tt_hbm.at[pl.ds(0, EMB_DIM), pl.ds(r0, TCHUNK)], in_x, sem)

    def out_desc(t, tr_x, sem):
        cc = wid + NW * t
        return pltpu.make_async_copy(
            tr_x, out_hbm.at[pl.ds(cc * TCHUNK * EMB_DIM, TCHUNK * EMB_DIM)],
            sem)

    rots = [(lane + s) % 16 for s in range(16)]
    stxs = [r * EMB_DIM + lane for r in rots]

    def transpose(in_x, tr_x):
        @plsc.parallel_loop(0, TCHUNK // 16, step=1, unroll=4)
        def tbody(m):
            for h in range(EMB_DIM // 16):
                for s in range(16):
                    vals = plsc.load_gather(
                        in_x, [lane + 16 * h, rots[s] + 16 * m])
                    plsc.store_scatter(
                        tr_x, [stxs[s] + (512 * m + 16 * h)], vals)

    bufs = ((in_a, tr_a, sem_ia, sem_oa), (in_b, tr_b, sem_ib, sem_ob))
    in_desc(0, in_a, sem_ia).start()
    in_desc(1, in_b, sem_ib).start()

    def body(tau, _):
        for par, (in_x, tr_x, sem_i, sem_o) in enumerate(bufs):
            t = 2 * tau + par

            @pl.when(wid + NW * t < NCHUNK)
            def _():
                in_desc(t, in_x, sem_i).wait()

                @pl.when(t >= 2)
                def _():
                    out_desc(t - 2, tr_x, sem_o).wait()

                transpose(in_x, tr_x)
                out_desc(t, tr_x, sem_o).start()

                @pl.when(wid + NW * (t + 2) < NCHUNK)
                def _():
                    in_desc(t + 2, in_x, sem_i).start()

        return 0

    tmax = (NCHUNK + NW - 1) // NW
    lax.fori_loop(0, (tmax + 2) // 2, body, 0)

    n_w = (NCHUNK - wid + NW - 1) // NW
    for par, (in_x, tr_x, sem_i, sem_o) in enumerate(bufs):
        t_last = par + 2 * ((n_w - par - 1) // 2)
        out_desc(t_last, tr_x, sem_o).wait()


@functools.partial(
    pl.kernel,
    mesh=plsc.VectorSubcoreMesh(core_axis_name="c", subcore_axis_name="s"),
    out_type=jax.ShapeDtypeStruct((BATCH, OUT_ROWS, EMB_DIM), jnp.float32),
    scratch_types=[
        pltpu.VMEM((BPW * SEQ_LEN,), jnp.int32),
        pltpu.VMEM((2 * GROUP, OUT_ROWS, EMB_DIM), jnp.float32),
        pltpu.SemaphoreType.DMA,
        pltpu.SemaphoreType.DMA,
    ],
    compiler_params=pltpu.CompilerParams(use_tc_tiling_on_sc=False),
)
def _emb_kernel(idx_hbm, table_hbm, empty_hbm, out_hbm,
                idx_v, rows_v, sem_g, sem_s):
    wid = lax.axis_index("s") * _NC + lax.axis_index("c")
    base = wid * BPW
    pltpu.sync_copy(idx_hbm.at[pl.ds(base * SEQ_LEN, BPW * SEQ_LEN)], idx_v)
    for d in range(2):
        for j in range(GROUP):
            pltpu.sync_copy(
                empty_hbm, rows_v.at[d * GROUP + j, pl.ds(0, 1)])

    def gather_copies(g, d, make_only):
        for j in range(GROUP):
            off = 0
            for n in CHUNKS:
                desc = pltpu.make_async_copy(
                    table_hbm.at[idx_v.at[pl.ds((g * GROUP + j) * SEQ_LEN
                                                + off, n)]],
                    rows_v.at[d * GROUP + j, pl.ds(1 + off, n)],
                    sem_g)
                if make_only:
                    desc.wait()
                else:
                    desc.start()
                off += n

    def gather_group(g, d):
        gather_copies(g, d, make_only=False)

    def drain_gathers(g, d):
        gather_copies(g, d, make_only=True)

    def store_desc(g, d):
        return pltpu.make_async_copy(
            rows_v.at[pl.ds(d * GROUP, GROUP)],
            out_hbm.at[pl.ds(base + g * GROUP, GROUP)],
            sem_s)

    gather_group(0, 0)

    def body(g, _):
        d = g % 2
        drain_gathers(g, d)

        @pl.when(g >= 1)
        def _():
            store_desc(g - 1, 1 - d).wait()

        store_desc(g, d).start()

        @pl.when(g + 1 < NGROUP)
        def _():
            gather_group(g + 1, 1 - d)

        return 0

    lax.fori_loop(0, NGROUP, body, 0)
    store_desc(NGROUP - 1, (NGROUP - 1) % 2).wait()


def kernel(sentence, table, empty_emb):
    idx = sentence[:, ::-1].astype(jnp.int32).reshape(-1)
    tail = table[NCHUNK * TCHUNK:].reshape(-1)
    table_lin = _detile_kernel(table.T, tail)
    table_rm = table_lin.reshape(VPAD, EMB_DIM)
    return _emb_kernel(idx, table_rm, empty_emb)


# speedup vs baseline: 1.8033x; 1.4077x over previous
import functools

import jax
import jax.numpy as jnp
from jax import lax
from jax.experimental import pallas as pl
from jax.experimental.pallas import tpu as pltpu
from jax.experimental.pallas import tpu_sc as plsc

BATCH = 1024
SEQ_LEN = 200
EMB_DIM = 32
VOCAB = 1000000
OUT_ROWS = SEQ_LEN + 1
CHUNKS = (104, 96)
GROUP = 8

_info = plsc.get_sparse_core_info()
_NC, _NS = _info.num_cores, _info.num_subcores
NW = _NC * _NS
BPW = BATCH // NW
NGROUP = BPW // GROUP
BLOCK_ROWS = GROUP * OUT_ROWS

TCHUNK = 256
NCHUNK = VOCAB // TCHUNK
TAIL = VOCAB - NCHUNK * TCHUNK
VPAD = VOCAB + (-VOCAB % TCHUNK)
TWORDS = VPAD * EMB_DIM


@functools.partial(
    pl.kernel,
    mesh=plsc.VectorSubcoreMesh(core_axis_name="c", subcore_axis_name="s"),
    out_type=jax.ShapeDtypeStruct((TWORDS,), jnp.float32),
    scratch_types=[
        pltpu.VMEM((EMB_DIM, TCHUNK), jnp.float32),
        pltpu.VMEM((EMB_DIM, TCHUNK), jnp.float32),
        pltpu.VMEM((TCHUNK * EMB_DIM,), jnp.float32),
        pltpu.VMEM((TCHUNK * EMB_DIM,), jnp.float32),
        pltpu.VMEM((TAIL * EMB_DIM,), jnp.float32),
        pltpu.SemaphoreType.DMA,
        pltpu.SemaphoreType.DMA,
        pltpu.SemaphoreType.DMA,
        pltpu.SemaphoreType.DMA,
    ],
    compiler_params=pltpu.CompilerParams(use_tc_tiling_on_sc=True,
                                        needs_layout_passes=False),
)
def _detile_kernel(tt_hbm, tail_hbm, out_hbm,
                   in_a, in_b, tr_a, tr_b, tail_v,
                   sem_ia, sem_ib, sem_oa, sem_ob):
    wid = lax.axis_index("s") * _NC + lax.axis_index("c")
    lane = lax.iota(jnp.int32, 16)

    @pl.when(wid == 0)
    def _():
        pltpu.sync_copy(tail_hbm, tail_v)
        pltpu.sync_copy(tail_v,
                        out_hbm.at[pl.ds(NCHUNK * TCHUNK * EMB_DIM,
                                         TAIL * EMB_DIM)])

    def in_descs(t, in_x, sem):
        r0 = (wid + NW * t) * TCHUNK
        return [pltpu.make_async_copy(
                    tt_hbm.at[pl.ds(8 * a, 8), pl.ds(r0, TCHUNK)],
                    in_x.at[pl.ds(8 * a, 8)], sem)
                for a in range(EMB_DIM // 8)]

    def out_desc(t, tr_x, sem):
        cc = wid + NW * t
        return pltpu.make_async_copy(
            tr_x, out_hbm.at[pl.ds(cc * TCHUNK * EMB_DIM, TCHUNK * EMB_DIM)],
            sem)

    rots = [(lane + s) % 16 for s in range(16)]
    stxs = [r * EMB_DIM + lane for r in rots]

    def transpose(in_x, tr_x):
        @plsc.parallel_loop(0, TCHUNK // 16, step=1, unroll=8)
        def tbody(m):
            for h in range(EMB_DIM // 16):
                for s in range(16):
                    vals = plsc.load_gather(
                        in_x, [lane + 16 * h, rots[s] + 16 * m])
                    plsc.store_scatter(
                        tr_x, [stxs[s] + (512 * m + 16 * h)], vals)

    bufs = ((in_a, tr_a, sem_ia, sem_oa), (in_b, tr_b, sem_ib, sem_ob))
    for dsc in in_descs(0, in_a, sem_ia):
        dsc.start()
    for dsc in in_descs(1, in_b, sem_ib):
        dsc.start()

    def body(tau, _):
        for par, (in_x, tr_x, sem_i, sem_o) in enumerate(bufs):
            t = 2 * tau + par

            @pl.when(wid + NW * t < NCHUNK)
            def _():
                for dsc in in_descs(t, in_x, sem_i):
                    dsc.wait()

                @pl.when(t >= 2)
                def _():
                    out_desc(t - 2, tr_x, sem_o).wait()

                transpose(in_x, tr_x)
                out_desc(t, tr_x, sem_o).start()

                @pl.when(wid + NW * (t + 2) < NCHUNK)
                def _():
                    for dsc in in_descs(t + 2, in_x, sem_i):
                        dsc.start()

        return 0

    tmax = (NCHUNK + NW - 1) // NW
    lax.fori_loop(0, (tmax + 2) // 2, body, 0)

    n_w = (NCHUNK - wid + NW - 1) // NW
    for par, (in_x, tr_x, sem_i, sem_o) in enumerate(bufs):
        t_last = par + 2 * ((n_w - par - 1) // 2)
        out_desc(t_last, tr_x, sem_o).wait()


@functools.partial(
    pl.kernel,
    mesh=plsc.VectorSubcoreMesh(core_axis_name="c", subcore_axis_name="s"),
    out_type=jax.ShapeDtypeStruct((BATCH, OUT_ROWS, EMB_DIM), jnp.float32),
    scratch_types=[
        pltpu.VMEM((BPW * SEQ_LEN,), jnp.int32),
        pltpu.VMEM((2 * GROUP, OUT_ROWS, EMB_DIM), jnp.float32),
        pltpu.SemaphoreType.DMA,
        pltpu.SemaphoreType.DMA,
    ],
    compiler_params=pltpu.CompilerParams(use_tc_tiling_on_sc=False),
)
def _emb_kernel(idx_hbm, table_hbm, empty_hbm, out_hbm,
                idx_v, rows_v, sem_g, sem_s):
    wid = lax.axis_index("s") * _NC + lax.axis_index("c")
    base = wid * BPW
    pltpu.sync_copy(idx_hbm.at[pl.ds(base * SEQ_LEN, BPW * SEQ_LEN)], idx_v)
    for d in range(2):
        for j in range(GROUP):
            pltpu.sync_copy(
                empty_hbm, rows_v.at[d * GROUP + j, pl.ds(0, 1)])

    def gather_copies(g, d, make_only):
        for j in range(GROUP):
            off = 0
            for n in CHUNKS:
                desc = pltpu.make_async_copy(
                    table_hbm.at[idx_v.at[pl.ds((g * GROUP + j) * SEQ_LEN
                                                + off, n)]],
                    rows_v.at[d * GROUP + j, pl.ds(1 + off, n)],
                    sem_g)
                if make_only:
                    desc.wait()
                else:
                    desc.start()
                off += n

    def gather_group(g, d):
        gather_copies(g, d, make_only=False)

    def drain_gathers(g, d):
        gather_copies(g, d, make_only=True)

    def store_desc(g, d):
        return pltpu.make_async_copy(
            rows_v.at[pl.ds(d * GROUP, GROUP)],
            out_hbm.at[pl.ds(base + g * GROUP, GROUP)],
            sem_s)

    gather_group(0, 0)

    def body(g, _):
        d = g % 2
        drain_gathers(g, d)

        @pl.when(g >= 1)
        def _():
            store_desc(g - 1, 1 - d).wait()

        store_desc(g, d).start()

        @pl.when(g + 1 < NGROUP)
        def _():
            gather_group(g + 1, 1 - d)

        return 0

    lax.fori_loop(0, NGROUP, body, 0)
    store_desc(NGROUP - 1, (NGROUP - 1) % 2).wait()


def kernel(sentence, table, empty_emb):
    idx = sentence[:, ::-1].astype(jnp.int32).reshape(-1)
    tail = table[NCHUNK * TCHUNK:].reshape(-1)
    table_lin = _detile_kernel(table.T, tail)
    table_rm = table_lin.reshape(VPAD, EMB_DIM)
    return _emb_kernel(idx, table_rm, empty_emb)


# speedup vs baseline: 1.8524x; 1.0272x over previous
import functools

import jax
import jax.numpy as jnp
from jax import lax
from jax.experimental import pallas as pl
from jax.experimental.pallas import tpu as pltpu
from jax.experimental.pallas import tpu_sc as plsc

BATCH = 1024
SEQ_LEN = 200
EMB_DIM = 32
VOCAB = 1000000
OUT_ROWS = SEQ_LEN + 1
CHUNKS = (104, 96)
GROUP = 8

_info = plsc.get_sparse_core_info()
_NC, _NS = _info.num_cores, _info.num_subcores
NW = _NC * _NS
BPW = BATCH // NW
NGROUP = BPW // GROUP
BLOCK_ROWS = GROUP * OUT_ROWS

TCHUNK = 512
NCHUNK = VOCAB // TCHUNK
TAIL = VOCAB - NCHUNK * TCHUNK
VPAD = VOCAB + (-VOCAB % TCHUNK)
TWORDS = VPAD * EMB_DIM


@functools.partial(
    pl.kernel,
    mesh=plsc.VectorSubcoreMesh(core_axis_name="c", subcore_axis_name="s"),
    out_type=jax.ShapeDtypeStruct((TWORDS,), jnp.float32),
    scratch_types=[
        pltpu.VMEM((EMB_DIM, TCHUNK), jnp.float32),
        pltpu.VMEM((EMB_DIM, TCHUNK), jnp.float32),
        pltpu.VMEM((TCHUNK * EMB_DIM,), jnp.float32),
        pltpu.VMEM((TCHUNK * EMB_DIM,), jnp.float32),
        pltpu.VMEM((TAIL * EMB_DIM,), jnp.float32),
        pltpu.SemaphoreType.DMA,
        pltpu.SemaphoreType.DMA,
        pltpu.SemaphoreType.DMA,
        pltpu.SemaphoreType.DMA,
    ],
    compiler_params=pltpu.CompilerParams(use_tc_tiling_on_sc=True,
                                        needs_layout_passes=False),
)
def _detile_kernel(tt_hbm, tail_hbm, out_hbm,
                   in_a, in_b, tr_a, tr_b, tail_v,
                   sem_ia, sem_ib, sem_oa, sem_ob):
    wid = lax.axis_index("s") * _NC + lax.axis_index("c")
    lane = lax.iota(jnp.int32, 16)

    @pl.when(wid == 0)
    def _():
        pltpu.sync_copy(tail_hbm, tail_v)
        pltpu.sync_copy(tail_v,
                        out_hbm.at[pl.ds(NCHUNK * TCHUNK * EMB_DIM,
                                         TAIL * EMB_DIM)])

    def in_descs(t, in_x, sem):
        r0 = (wid + NW * t) * TCHUNK
        return [pltpu.make_async_copy(
                    tt_hbm.at[pl.ds(8 * a, 8), pl.ds(r0, TCHUNK)],
                    in_x.at[pl.ds(8 * a, 8)], sem)
                for a in range(EMB_DIM // 8)]

    def out_desc(t, tr_x, sem):
        cc = wid + NW * t
        return pltpu.make_async_copy(
            tr_x, out_hbm.at[pl.ds(cc * TCHUNK * EMB_DIM, TCHUNK * EMB_DIM)],
            sem)

    rots = [(lane + s) % 16 for s in range(16)]
    stxs = [r * EMB_DIM + lane for r in rots]

    def transpose(in_x, tr_x):
        @plsc.parallel_loop(0, TCHUNK // 16, step=1, unroll=8)
        def tbody(m):
            for h in range(EMB_DIM // 16):
                for s in range(16):
                    vals = plsc.load_gather(
                        in_x, [lane + 16 * h, rots[s] + 16 * m])
                    plsc.store_scatter(
                        tr_x, [stxs[s] + (512 * m + 16 * h)], vals)

    bufs = ((in_a, tr_a, sem_ia, sem_oa), (in_b, tr_b, sem_ib, sem_ob))
    for dsc in in_descs(0, in_a, sem_ia):
        dsc.start()
    for dsc in in_descs(1, in_b, sem_ib):
        dsc.start()

    def body(tau, _):
        for par, (in_x, tr_x, sem_i, sem_o) in enumerate(bufs):
            t = 2 * tau + par

            @pl.when(wid + NW * t < NCHUNK)
            def _():
                for dsc in in_descs(t, in_x, sem_i):
                    dsc.wait()

                @pl.when(t >= 2)
                def _():
                    out_desc(t - 2, tr_x, sem_o).wait()

                transpose(in_x, tr_x)
                out_desc(t, tr_x, sem_o).start()

                @pl.when(wid + NW * (t + 2) < NCHUNK)
                def _():
                    for dsc in in_descs(t + 2, in_x, sem_i):
                        dsc.start()

        return 0

    tmax = (NCHUNK + NW - 1) // NW
    lax.fori_loop(0, (tmax + 2) // 2, body, 0)

    n_w = (NCHUNK - wid + NW - 1) // NW
    for par, (in_x, tr_x, sem_i, sem_o) in enumerate(bufs):
        t_last = par + 2 * ((n_w - par - 1) // 2)
        out_desc(t_last, tr_x, sem_o).wait()


@functools.partial(
    pl.kernel,
    mesh=plsc.VectorSubcoreMesh(core_axis_name="c", subcore_axis_name="s"),
    out_type=jax.ShapeDtypeStruct((BATCH, OUT_ROWS, EMB_DIM), jnp.float32),
    scratch_types=[
        pltpu.VMEM((BPW * SEQ_LEN,), jnp.int32),
        pltpu.VMEM((2 * GROUP, OUT_ROWS, EMB_DIM), jnp.float32),
        pltpu.SemaphoreType.DMA,
        pltpu.SemaphoreType.DMA,
    ],
    compiler_params=pltpu.CompilerParams(use_tc_tiling_on_sc=False),
)
def _emb_kernel(idx_hbm, table_hbm, empty_hbm, out_hbm,
                idx_v, rows_v, sem_g, sem_s):
    wid = lax.axis_index("s") * _NC + lax.axis_index("c")
    base = wid * BPW
    pltpu.sync_copy(idx_hbm.at[pl.ds(base * SEQ_LEN, BPW * SEQ_LEN)], idx_v)
    for d in range(2):
        for j in range(GROUP):
            pltpu.sync_copy(
                empty_hbm, rows_v.at[d * GROUP + j, pl.ds(0, 1)])

    def gather_copies(g, d, make_only):
        for j in range(GROUP):
            off = 0
            for n in CHUNKS:
                desc = pltpu.make_async_copy(
                    table_hbm.at[idx_v.at[pl.ds((g * GROUP + j) * SEQ_LEN
                                                + off, n)]],
                    rows_v.at[d * GROUP + j, pl.ds(1 + off, n)],
                    sem_g)
                if make_only:
                    desc.wait()
                else:
                    desc.start()
                off += n

    def gather_group(g, d):
        gather_copies(g, d, make_only=False)

    def drain_gathers(g, d):
        gather_copies(g, d, make_only=True)

    def store_desc(g, d):
        return pltpu.make_async_copy(
            rows_v.at[pl.ds(d * GROUP, GROUP)],
            out_hbm.at[pl.ds(base + g * GROUP, GROUP)],
            sem_s)

    gather_group(0, 0)

    def body(g, _):
        d = g % 2
        drain_gathers(g, d)

        @pl.when(g >= 1)
        def _():
            store_desc(g - 1, 1 - d).wait()

        store_desc(g, d).start()

        @pl.when(g + 1 < NGROUP)
        def _():
            gather_group(g + 1, 1 - d)

        return 0

    lax.fori_loop(0, NGROUP, body, 0)
    store_desc(NGROUP - 1, (NGROUP - 1) % 2).wait()


def kernel(sentence, table, empty_emb):
    idx = sentence[:, ::-1].astype(jnp.int32).reshape(-1)
    tail = table[NCHUNK * TCHUNK:].reshape(-1)
    table_lin = _detile_kernel(table.T, tail)
    table_rm = table_lin.reshape(VPAD, EMB_DIM)
    return _emb_kernel(idx, table_rm, empty_emb)
